# baseline (speedup 1.0000x reference)
import jax
import jax.numpy as jnp
from jax.experimental import pallas as pl
from jax.experimental.pallas import tpu as pltpu

N = 524288
C = 32
BLK = 8192
GRID = N // BLK


def _loss_body(x_ref, g_ref, out_ref):
    i = pl.program_id(0)

    @pl.when(i == 0)
    def _():
        out_ref[0, 0] = 0.0

    x = g_ref[...] - x_ref[...]
    loss = jnp.maximum(x, 0.0) - x + jnp.log1p(jnp.exp(-jnp.abs(x)))
    out_ref[0, 0] += jnp.sum(loss)


def kernel(true_data, fake_data):
    slx = jnp.sort(true_data, axis=0)
    slg = jnp.sort(fake_data, axis=0)
    total = pl.pallas_call(
        _loss_body,
        grid=(GRID,),
        in_specs=[
            pl.BlockSpec((BLK, C), lambda i: (i, 0)),
            pl.BlockSpec((BLK, C), lambda i: (i, 0)),
        ],
        out_specs=pl.BlockSpec(memory_space=pltpu.SMEM),
        out_shape=jax.ShapeDtypeStruct((1, 1), jnp.float32),
    )(slx, slg)
    return total[0, 0] / (N * C)
